# baseline (speedup 1.0000x reference)
import functools
import jax
import jax.numpy as jnp
from jax import lax
from jax.experimental import pallas as pl
from jax.experimental.pallas import tpu as pltpu
from jax.experimental.pallas import tpu_sc as plsc

N = 10000
M = 2000
E = 320000
E2 = 320000
D = 128
H = 4
HID = 128
HD = D // H
NPAD = 10240
TILE_E = 512
TILE_N = 512


def _gelu(x):
    return 0.5 * x * (1.0 + lax.erf(x * 0.7071067811865476))


def _ln_kernel(x_ref, s_ref, b_ref, o_ref):
    x = x_ref[...]
    mu = jnp.mean(x, axis=-1, keepdims=True)
    var = jnp.mean((x - mu) ** 2, axis=-1, keepdims=True)
    o_ref[...] = (x - mu) / jnp.sqrt(var + 1e-5) * s_ref[...] + b_ref[...]


def _layer_norm_tc(x, scale, bias, tile):
    rows = x.shape[0]
    grid = rows // tile
    return pl.pallas_call(
        _ln_kernel,
        grid=(grid,),
        in_specs=[
            pl.BlockSpec((tile, D), lambda i: (i, 0)),
            pl.BlockSpec((1, D), lambda i: (0, 0)),
            pl.BlockSpec((1, D), lambda i: (0, 0)),
        ],
        out_specs=pl.BlockSpec((tile, D), lambda i: (i, 0)),
        out_shape=jax.ShapeDtypeStruct((rows, D), jnp.float32),
    )(x, scale.reshape(1, D), bias.reshape(1, D))


def _inter_edge_kernel(sc_ref, tc_ref, wsrc_ref, wtgt_ref, bsum_ref,
                       attn_ref, blk_ref, wmsg_ref, bmsg_ref, kron_ref,
                       tile16_ref, msgw_ref, exrow_ref):
    scv = sc_ref[...].astype(jnp.bfloat16)
    tcv = tc_ref[...].astype(jnp.bfloat16)
    h = (jnp.dot(scv, wsrc_ref[...], preferred_element_type=jnp.float32)
         + jnp.dot(tcv, wtgt_ref[...], preferred_element_type=jnp.float32)
         + bsum_ref[...])
    h = _gelu(h)
    logits = jnp.dot(h * attn_ref[...], blk_ref[...],
                     preferred_element_type=jnp.float32)
    ex = jnp.exp(logits)
    msg = jnp.dot(scv, wmsg_ref[...], preferred_element_type=jnp.float32) + bmsg_ref[...]
    scale = jnp.dot(ex[:, :H], kron_ref[...], preferred_element_type=jnp.float32)
    msgw_ref[...] = msg * scale
    exrow_ref[...] = jnp.dot(ex, tile16_ref[...], preferred_element_type=jnp.float32)


def _intra_edge_kernel(ps_ref, pd_ref, br_ref, wsrc_ref, wtgt_ref, wbrg_ref,
                       bsum_ref, attn_ref, blk_ref, wm1_ref, wm2_ref,
                       bmsg_ref, kron_ref, tile16_ref, msgw_ref, exrow_ref):
    psv = ps_ref[...].astype(jnp.bfloat16)
    brv = br_ref[...].astype(jnp.bfloat16)
    h = (jnp.dot(psv, wsrc_ref[...], preferred_element_type=jnp.float32)
         + jnp.dot(pd_ref[...].astype(jnp.bfloat16), wtgt_ref[...],
                   preferred_element_type=jnp.float32)
         + jnp.dot(brv, wbrg_ref[...], preferred_element_type=jnp.float32)
         + bsum_ref[...])
    h = _gelu(h)
    logits = jnp.dot(h * attn_ref[...], blk_ref[...],
                     preferred_element_type=jnp.float32)
    ex = jnp.exp(logits)
    msg = (jnp.dot(psv, wm1_ref[...], preferred_element_type=jnp.float32)
           + jnp.dot(brv, wm2_ref[...], preferred_element_type=jnp.float32)
           + bmsg_ref[...])
    scale = jnp.dot(ex[:, :H], kron_ref[...], preferred_element_type=jnp.float32)
    msgw_ref[...] = msg * scale
    exrow_ref[...] = jnp.dot(ex, tile16_ref[...], preferred_element_type=jnp.float32)


def _final_kernel(nh_ref, u1_ref, su1_ref, u2_ref, su2_ref, wnp_ref, bnp_ref,
                  wo1_ref, bo1_ref, wo2_ref, bo2_ref, kron_ref, o_ref):
    kron = kron_ref[...]
    s1 = (su1_ref[0] + su1_ref[1])[:, :H]
    s2 = (su2_ref[0] + su2_ref[1])[:, :H]
    d1 = jnp.dot(s1, kron, preferred_element_type=jnp.float32) + 1e-16
    d2 = jnp.dot(s2, kron, preferred_element_type=jnp.float32) + 1e-16
    g1 = (u1_ref[0] + u1_ref[1]) / d1
    g2 = (u2_ref[0] + u2_ref[1]) / d2
    out = (jnp.dot(nh_ref[...], wnp_ref[...], preferred_element_type=jnp.float32)
           + bnp_ref[...]
           + jnp.dot(g1, wo1_ref[...], preferred_element_type=jnp.float32)
           + bo1_ref[...]
           + jnp.dot(g2, wo2_ref[...], preferred_element_type=jnp.float32)
           + bo2_ref[...])
    o_ref[...] = out


SC_NC = 2
SC_NS = 16
SC_NW = SC_NC * SC_NS


def _gather_all_sc(eh, nh, edge_idx, node_idx, pair_src, pair_dst):
    B = E
    b_per_w = B // SC_NW
    GB = 80
    nblk = b_per_w // GB
    mesh = plsc.VectorSubcoreMesh(core_axis_name="c", subcore_axis_name="s",
                                  num_cores=SC_NC, num_subcores=SC_NS)
    row_t = jax.ShapeDtypeStruct((B, D), jnp.float32)

    @functools.partial(
        pl.kernel, mesh=mesh,
        out_type=[row_t] * 4,
        scratch_types=(
            [pltpu.VMEM((b_per_w,), jnp.int32)] * 4
            + [pltpu.VMEM((GB, D), jnp.float32)] * 4
            + [pltpu.SemaphoreType.DMA] * 4
        ),
    )
    def k(eh_hbm, nh_hbm, ie_hbm, in_hbm, is_hbm, id_hbm,
          o1, o2, o3, o4, i1, i2, i3, i4, r1, r2, r3, r4, s1, s2, s3, s4):
        wid = lax.axis_index("s") * SC_NC + lax.axis_index("c")
        base = wid * b_per_w
        pltpu.sync_copy(ie_hbm.at[pl.ds(base, b_per_w)], i1)
        pltpu.sync_copy(in_hbm.at[pl.ds(base, b_per_w)], i2)
        pltpu.sync_copy(is_hbm.at[pl.ds(base, b_per_w)], i3)
        pltpu.sync_copy(id_hbm.at[pl.ds(base, b_per_w)], i4)

        def body(j, carry):
            off = j * GB
            sl = pl.ds(off, GB)
            osl = pl.ds(base + off, GB)
            c1 = pltpu.async_copy(eh_hbm.at[i1.at[sl]], r1, s1)
            c2 = pltpu.async_copy(nh_hbm.at[i2.at[sl]], r2, s2)
            c3 = pltpu.async_copy(nh_hbm.at[i3.at[sl]], r3, s3)
            c4 = pltpu.async_copy(nh_hbm.at[i4.at[sl]], r4, s4)
            c1.wait()
            pltpu.sync_copy(r1, o1.at[osl])
            c2.wait()
            pltpu.sync_copy(r2, o2.at[osl])
            c3.wait()
            pltpu.sync_copy(r3, o3.at[osl])
            c4.wait()
            pltpu.sync_copy(r4, o4.at[osl])
            return carry

        lax.fori_loop(0, nblk, body, 0)

    return k(eh, nh, edge_idx, node_idx, pair_src, pair_dst)


def _scatter_rows_sc(msgy, idx2d, zeros_u):
    B = msgy.shape[0]
    b_per_w = B // SC_NW
    RB = 80
    nblk = b_per_w // RB
    rows_per_s = NPAD // SC_NS
    mesh = plsc.VectorSubcoreMesh(core_axis_name="c", subcore_axis_name="s",
                                  num_cores=SC_NC, num_subcores=SC_NS)

    @functools.partial(
        pl.kernel, mesh=mesh,
        out_type=jax.ShapeDtypeStruct((SC_NC, NPAD, D), jnp.float32),
        scratch_types=[
            pltpu.VMEM((nblk, RB), jnp.int32),
            pltpu.VMEM((RB, D), jnp.float32),
            pltpu.VMEM_SHARED((NPAD, D), jnp.float32),
        ],
    )
    def k(msgy_hbm, idx2d_hbm, zu_hbm, u_out, idx2_v, m_v, uacc):
        cid = lax.axis_index("c")
        sid = lax.axis_index("s")
        wid = sid * SC_NC + cid
        base = wid * b_per_w

        pltpu.sync_copy(idx2d_hbm.at[wid], idx2_v)
        pltpu.sync_copy(zu_hbm, uacc.at[pl.ds(sid * rows_per_s, rows_per_s)])
        plsc.subcore_barrier()

        def row_body(j, carry):
            pltpu.sync_copy(msgy_hbm.at[pl.ds(base + j * RB, RB)], m_v)
            pltpu.sync_copy(m_v, uacc.at[idx2_v.at[j]], add=True)
            return carry

        lax.fori_loop(0, nblk, row_body, 0)
        plsc.subcore_barrier()

        pltpu.sync_copy(uacc.at[pl.ds(sid * rows_per_s, rows_per_s)],
                        u_out.at[cid, pl.ds(sid * rows_per_s, rows_per_s)])

    return k(msgy, idx2d, zeros_u)


def _full_spec(shape):
    nd = len(shape)
    return pl.BlockSpec(shape, lambda i: (0,) * nd)


def kernel(node_features, edge_features, bridge_features, params, node_idx, edge_idx, pair_src, pair_dst):
    p = params
    f32 = jnp.float32
    bf = lambda a: a.astype(jnp.bfloat16)

    hid_ids = jnp.arange(H * HID, dtype=jnp.int32) // HID
    blk = (hid_ids[:, None] == jnp.arange(8, dtype=jnp.int32)[None, :]).astype(f32)
    hd_ids = jnp.arange(D, dtype=jnp.int32) // HD
    kron = (jnp.arange(H, dtype=jnp.int32)[:, None] == hd_ids[None, :]).astype(f32)
    col_mod8 = jnp.arange(D, dtype=jnp.int32) % 8
    tile16 = (jnp.arange(8, dtype=jnp.int32)[:, None] == col_mod8[None, :]).astype(f32)

    nf_pad = jnp.pad(node_features, ((0, NPAD - N), (0, 0)))
    nh_pad = _layer_norm_tc(nf_pad, p['ln_n_s'], p['ln_n_b'], 512)
    nh = nh_pad[:N]
    eh = _layer_norm_tc(edge_features, p['ln_e_s'], p['ln_e_b'], 400)

    sc_i, tc_i, ps, pd = _gather_all_sc(eh, nh, edge_idx, node_idx,
                                        pair_src, pair_dst)

    bsum_i = (p['i_bsrc'] + p['i_btgt']).reshape(1, H * HID)
    attn_i = p['i_attn'].reshape(1, H * HID)
    grid_e = E // TILE_E
    msgw1, exrow1 = pl.pallas_call(
        _inter_edge_kernel,
        grid=(grid_e,),
        in_specs=[
            pl.BlockSpec((TILE_E, D), lambda i: (i, 0)),
            pl.BlockSpec((TILE_E, D), lambda i: (i, 0)),
            _full_spec((D, H * HID)),
            _full_spec((D, H * HID)),
            _full_spec((1, H * HID)),
            _full_spec((1, H * HID)),
            _full_spec((H * HID, 8)),
            _full_spec((D, D)),
            _full_spec((1, D)),
            _full_spec((H, D)),
            _full_spec((8, D)),
        ],
        out_specs=[
            pl.BlockSpec((TILE_E, D), lambda i: (i, 0)),
            pl.BlockSpec((TILE_E, D), lambda i: (i, 0)),
        ],
        out_shape=[
            jax.ShapeDtypeStruct((E, D), f32),
            jax.ShapeDtypeStruct((E, D), f32),
        ],
    )(sc_i, tc_i, bf(p['i_Wsrc']), bf(p['i_Wtgt']), bsum_i, attn_i, blk,
      bf(p['i_Wmsg']), p['i_bmsg'].reshape(1, D), kron, tile16)

    bsum_a = (p['a_bsrc'] + p['a_btgt'] + p['a_bbrg']).reshape(1, H * HID)
    attn_a = p['a_attn'].reshape(1, H * HID)
    grid_e2 = E2 // TILE_E
    msgw2, exrow2 = pl.pallas_call(
        _intra_edge_kernel,
        grid=(grid_e2,),
        in_specs=[
            pl.BlockSpec((TILE_E, D), lambda i: (i, 0)),
            pl.BlockSpec((TILE_E, D), lambda i: (i, 0)),
            pl.BlockSpec((TILE_E, D), lambda i: (i, 0)),
            _full_spec((D, H * HID)),
            _full_spec((D, H * HID)),
            _full_spec((D, H * HID)),
            _full_spec((1, H * HID)),
            _full_spec((1, H * HID)),
            _full_spec((H * HID, 8)),
            _full_spec((D, D)),
            _full_spec((D, D)),
            _full_spec((1, D)),
            _full_spec((H, D)),
            _full_spec((8, D)),
        ],
        out_specs=[
            pl.BlockSpec((TILE_E, D), lambda i: (i, 0)),
            pl.BlockSpec((TILE_E, D), lambda i: (i, 0)),
        ],
        out_shape=[
            jax.ShapeDtypeStruct((E2, D), f32),
            jax.ShapeDtypeStruct((E2, D), f32),
        ],
    )(ps, pd, bridge_features, bf(p['a_Wsrc']), bf(p['a_Wtgt']), bf(p['a_Wbrg']),
      bsum_a, attn_a, blk, bf(p['a_Wmsg'][:D]), bf(p['a_Wmsg'][D:]),
      p['a_bmsg'].reshape(1, D), kron, tile16)

    zu = jnp.zeros((NPAD // SC_NS, D), f32)
    idx2d_1 = node_idx.reshape(SC_NW, -1, 80)
    idx2d_2 = pair_dst.reshape(SC_NW, -1, 80)
    u1 = _scatter_rows_sc(msgw1, idx2d_1, zu)
    su1 = _scatter_rows_sc(exrow1, idx2d_1, zu)
    u2 = _scatter_rows_sc(msgw2, idx2d_2, zu)
    su2 = _scatter_rows_sc(exrow2, idx2d_2, zu)

    grid_n = NPAD // TILE_N
    out = pl.pallas_call(
        _final_kernel,
        grid=(grid_n,),
        in_specs=[
            pl.BlockSpec((TILE_N, D), lambda i: (i, 0)),
            pl.BlockSpec((SC_NC, TILE_N, D), lambda i: (0, i, 0)),
            pl.BlockSpec((SC_NC, TILE_N, D), lambda i: (0, i, 0)),
            pl.BlockSpec((SC_NC, TILE_N, D), lambda i: (0, i, 0)),
            pl.BlockSpec((SC_NC, TILE_N, D), lambda i: (0, i, 0)),
            _full_spec((D, D)),
            _full_spec((1, D)),
            _full_spec((D, D)),
            _full_spec((1, D)),
            _full_spec((D, D)),
            _full_spec((1, D)),
            _full_spec((H, D)),
        ],
        out_specs=pl.BlockSpec((TILE_N, D), lambda i: (i, 0)),
        out_shape=jax.ShapeDtypeStruct((NPAD, D), f32),
    )(nh_pad, u1, su1, u2, su2, p['Wnp'], p['bnp'].reshape(1, D),
      p['i_Wout'], p['i_bout'].reshape(1, D),
      p['a_Wout'], p['a_bout'].reshape(1, D), kron)

    return out[:N]


# speedup vs baseline: 1.2109x; 1.2109x over previous
import functools
import jax
import jax.numpy as jnp
from jax import lax
from jax.experimental import pallas as pl
from jax.experimental.pallas import tpu as pltpu
from jax.experimental.pallas import tpu_sc as plsc

N = 10000
M = 2000
E = 320000
E2 = 320000
D = 128
H = 4
HID = 128
HD = D // H
NPAD = 10240
TILE_E = 512
TILE_N = 512


def _gelu(x):
    return 0.5 * x * (1.0 + lax.erf(x * 0.7071067811865476))


def _ln_kernel(x_ref, s_ref, b_ref, o_ref):
    x = x_ref[...]
    mu = jnp.mean(x, axis=-1, keepdims=True)
    var = jnp.mean((x - mu) ** 2, axis=-1, keepdims=True)
    o_ref[...] = (x - mu) / jnp.sqrt(var + 1e-5) * s_ref[...] + b_ref[...]


def _layer_norm_tc(x, scale, bias, tile):
    rows = x.shape[0]
    grid = rows // tile
    return pl.pallas_call(
        _ln_kernel,
        grid=(grid,),
        in_specs=[
            pl.BlockSpec((tile, D), lambda i: (i, 0)),
            pl.BlockSpec((1, D), lambda i: (0, 0)),
            pl.BlockSpec((1, D), lambda i: (0, 0)),
        ],
        out_specs=pl.BlockSpec((tile, D), lambda i: (i, 0)),
        out_shape=jax.ShapeDtypeStruct((rows, D), jnp.float32),
    )(x, scale.reshape(1, D), bias.reshape(1, D))


def _inter_edge_kernel(sc_ref, tc_ref, wsrc_ref, wtgt_ref, bsum_ref,
                       attn_ref, blk_ref, wmsg_ref, bmsg_ref, kron_ref,
                       tile16_ref, msgw_ref, exrow_ref):
    scv = sc_ref[...].astype(jnp.bfloat16)
    tcv = tc_ref[...].astype(jnp.bfloat16)
    h = (jnp.dot(scv, wsrc_ref[...], preferred_element_type=jnp.float32)
         + jnp.dot(tcv, wtgt_ref[...], preferred_element_type=jnp.float32)
         + bsum_ref[...])
    h = _gelu(h)
    logits = jnp.dot(h * attn_ref[...], blk_ref[...],
                     preferred_element_type=jnp.float32)
    ex = jnp.exp(logits)
    msg = jnp.dot(scv, wmsg_ref[...], preferred_element_type=jnp.float32) + bmsg_ref[...]
    scale = jnp.dot(ex[:, :H], kron_ref[...], preferred_element_type=jnp.float32)
    msgw_ref[...] = msg * scale
    exrow_ref[...] = jnp.dot(ex, tile16_ref[...], preferred_element_type=jnp.float32)


def _intra_edge_kernel(ps_ref, pd_ref, br_ref, wsrc_ref, wtgt_ref, wbrg_ref,
                       bsum_ref, attn_ref, blk_ref, wm1_ref, wm2_ref,
                       bmsg_ref, kron_ref, tile16_ref, msgw_ref, exrow_ref):
    psv = ps_ref[...].astype(jnp.bfloat16)
    brv = br_ref[...].astype(jnp.bfloat16)
    h = (jnp.dot(psv, wsrc_ref[...], preferred_element_type=jnp.float32)
         + jnp.dot(pd_ref[...].astype(jnp.bfloat16), wtgt_ref[...],
                   preferred_element_type=jnp.float32)
         + jnp.dot(brv, wbrg_ref[...], preferred_element_type=jnp.float32)
         + bsum_ref[...])
    h = _gelu(h)
    logits = jnp.dot(h * attn_ref[...], blk_ref[...],
                     preferred_element_type=jnp.float32)
    ex = jnp.exp(logits)
    msg = (jnp.dot(psv, wm1_ref[...], preferred_element_type=jnp.float32)
           + jnp.dot(brv, wm2_ref[...], preferred_element_type=jnp.float32)
           + bmsg_ref[...])
    scale = jnp.dot(ex[:, :H], kron_ref[...], preferred_element_type=jnp.float32)
    msgw_ref[...] = msg * scale
    exrow_ref[...] = jnp.dot(ex, tile16_ref[...], preferred_element_type=jnp.float32)


def _final_kernel(nh_ref, u1_ref, su1_ref, u2_ref, su2_ref, wnp_ref, bnp_ref,
                  wo1_ref, bo1_ref, wo2_ref, bo2_ref, kron_ref, o_ref):
    kron = kron_ref[...]
    s1 = (su1_ref[0] + su1_ref[1])[:, :H]
    s2 = (su2_ref[0] + su2_ref[1])[:, :H]
    d1 = jnp.dot(s1, kron, preferred_element_type=jnp.float32) + 1e-16
    d2 = jnp.dot(s2, kron, preferred_element_type=jnp.float32) + 1e-16
    g1 = (u1_ref[0] + u1_ref[1]) / d1
    g2 = (u2_ref[0] + u2_ref[1]) / d2
    out = (jnp.dot(nh_ref[...], wnp_ref[...], preferred_element_type=jnp.float32)
           + bnp_ref[...]
           + jnp.dot(g1, wo1_ref[...], preferred_element_type=jnp.float32)
           + bo1_ref[...]
           + jnp.dot(g2, wo2_ref[...], preferred_element_type=jnp.float32)
           + bo2_ref[...])
    o_ref[...] = out


SC_NC = 2
SC_NS = 16
SC_NW = SC_NC * SC_NS


def _gather_rows_sc(table, idx):
    B = idx.shape[0]
    b_per_w = B // SC_NW
    GB = 400
    nblk = b_per_w // GB
    mesh = plsc.VectorSubcoreMesh(core_axis_name="c", subcore_axis_name="s",
                                  num_cores=SC_NC, num_subcores=SC_NS)

    @functools.partial(
        pl.kernel, mesh=mesh,
        out_type=jax.ShapeDtypeStruct((B, D), jnp.float32),
        scratch_types=[
            pltpu.VMEM((b_per_w,), jnp.int32),
            pltpu.VMEM((GB, D), jnp.float32),
            pltpu.SemaphoreType.DMA,
        ],
    )
    def k(table_hbm, idx_hbm, out_hbm, idx_v, rows_v, sem):
        wid = lax.axis_index("s") * SC_NC + lax.axis_index("c")
        base = wid * b_per_w
        pltpu.sync_copy(idx_hbm.at[pl.ds(base, b_per_w)], idx_v)

        def body(j, carry):
            off = j * GB
            pltpu.async_copy(table_hbm.at[idx_v.at[pl.ds(off, GB)]],
                             rows_v, sem).wait()
            pltpu.sync_copy(rows_v, out_hbm.at[pl.ds(base + off, GB)])
            return carry

        lax.fori_loop(0, nblk, body, 0)

    return k(table, idx)


def _scatter2_rows_sc(msgw, exrow, idx2d, zeros_u):
    B = msgw.shape[0]
    b_per_w = B // SC_NW
    RB = 80
    nblk = b_per_w // RB
    npair = (nblk - 1) // 2
    rows_per_s = NPAD // SC_NS
    mesh = plsc.VectorSubcoreMesh(core_axis_name="c", subcore_axis_name="s",
                                  num_cores=SC_NC, num_subcores=SC_NS)
    part_t = jax.ShapeDtypeStruct((SC_NC, NPAD, D), jnp.float32)

    @functools.partial(
        pl.kernel, mesh=mesh,
        out_type=[part_t, part_t],
        scratch_types=[
            pltpu.VMEM((nblk, RB), jnp.int32),
            pltpu.VMEM((RB, D), jnp.float32),
            pltpu.VMEM((RB, D), jnp.float32),
            pltpu.VMEM_SHARED((NPAD, D), jnp.float32),
            pltpu.SemaphoreType.DMA,
            pltpu.SemaphoreType.DMA,
        ],
    )
    def k(msgw_hbm, exrow_hbm, idx2d_hbm, zu_hbm, u_out, s_out,
          idx2_v, m_a, m_b, uacc, sem_a, sem_b):
        cid = lax.axis_index("c")
        sid = lax.axis_index("s")
        wid = sid * SC_NC + cid
        base = wid * b_per_w
        my_rows = pl.ds(sid * rows_per_s, rows_per_s)

        pltpu.sync_copy(idx2d_hbm.at[wid], idx2_v)
        pltpu.sync_copy(zu_hbm, uacc.at[my_rows])
        plsc.subcore_barrier()

        def scatter_stream(rows_hbm):
            pltpu.async_copy(rows_hbm.at[pl.ds(base, RB)], m_a, sem_a).wait()

            def pair_body(i, carry):
                j0 = 2 * i
                cb = pltpu.async_copy(
                    rows_hbm.at[pl.ds(base + (j0 + 1) * RB, RB)], m_b, sem_b)
                pltpu.sync_copy(m_a, uacc.at[idx2_v.at[j0]], add=True)
                cb.wait()
                ca = pltpu.async_copy(
                    rows_hbm.at[pl.ds(base + (j0 + 2) * RB, RB)], m_a, sem_a)
                pltpu.sync_copy(m_b, uacc.at[idx2_v.at[j0 + 1]], add=True)
                ca.wait()
                return carry

            lax.fori_loop(0, npair, pair_body, 0)
            pltpu.sync_copy(m_a, uacc.at[idx2_v.at[nblk - 1]], add=True)

        scatter_stream(msgw_hbm)
        plsc.subcore_barrier()
        pltpu.sync_copy(uacc.at[my_rows], u_out.at[cid, my_rows])
        pltpu.sync_copy(zu_hbm, uacc.at[my_rows])
        plsc.subcore_barrier()

        scatter_stream(exrow_hbm)
        plsc.subcore_barrier()
        pltpu.sync_copy(uacc.at[my_rows], s_out.at[cid, my_rows])

    return k(msgw, exrow, idx2d, zeros_u)


def _full_spec(shape):
    nd = len(shape)
    return pl.BlockSpec(shape, lambda i: (0,) * nd)


def kernel(node_features, edge_features, bridge_features, params, node_idx, edge_idx, pair_src, pair_dst):
    p = params
    f32 = jnp.float32
    bf = lambda a: a.astype(jnp.bfloat16)

    hid_ids = jnp.arange(H * HID, dtype=jnp.int32) // HID
    blk = (hid_ids[:, None] == jnp.arange(8, dtype=jnp.int32)[None, :]).astype(f32)
    hd_ids = jnp.arange(D, dtype=jnp.int32) // HD
    kron = (jnp.arange(H, dtype=jnp.int32)[:, None] == hd_ids[None, :]).astype(f32)
    col_mod8 = jnp.arange(D, dtype=jnp.int32) % 8
    tile16 = (jnp.arange(8, dtype=jnp.int32)[:, None] == col_mod8[None, :]).astype(f32)

    nf_pad = jnp.pad(node_features, ((0, NPAD - N), (0, 0)))
    nh_pad = _layer_norm_tc(nf_pad, p['ln_n_s'], p['ln_n_b'], 512)
    nh = nh_pad[:N]
    eh = _layer_norm_tc(edge_features, p['ln_e_s'], p['ln_e_b'], 400)

    sc_i = _gather_rows_sc(eh, edge_idx)
    tc_i = _gather_rows_sc(nh, node_idx)
    ps = _gather_rows_sc(nh, pair_src)
    pd = _gather_rows_sc(nh, pair_dst)

    bsum_i = (p['i_bsrc'] + p['i_btgt']).reshape(1, H * HID)
    attn_i = p['i_attn'].reshape(1, H * HID)
    grid_e = E // TILE_E
    msgw1, exrow1 = pl.pallas_call(
        _inter_edge_kernel,
        grid=(grid_e,),
        in_specs=[
            pl.BlockSpec((TILE_E, D), lambda i: (i, 0)),
            pl.BlockSpec((TILE_E, D), lambda i: (i, 0)),
            _full_spec((D, H * HID)),
            _full_spec((D, H * HID)),
            _full_spec((1, H * HID)),
            _full_spec((1, H * HID)),
            _full_spec((H * HID, 8)),
            _full_spec((D, D)),
            _full_spec((1, D)),
            _full_spec((H, D)),
            _full_spec((8, D)),
        ],
        out_specs=[
            pl.BlockSpec((TILE_E, D), lambda i: (i, 0)),
            pl.BlockSpec((TILE_E, D), lambda i: (i, 0)),
        ],
        out_shape=[
            jax.ShapeDtypeStruct((E, D), f32),
            jax.ShapeDtypeStruct((E, D), f32),
        ],
    )(sc_i, tc_i, bf(p['i_Wsrc']), bf(p['i_Wtgt']), bsum_i, attn_i, blk,
      bf(p['i_Wmsg']), p['i_bmsg'].reshape(1, D), kron, tile16)

    bsum_a = (p['a_bsrc'] + p['a_btgt'] + p['a_bbrg']).reshape(1, H * HID)
    attn_a = p['a_attn'].reshape(1, H * HID)
    grid_e2 = E2 // TILE_E
    msgw2, exrow2 = pl.pallas_call(
        _intra_edge_kernel,
        grid=(grid_e2,),
        in_specs=[
            pl.BlockSpec((TILE_E, D), lambda i: (i, 0)),
            pl.BlockSpec((TILE_E, D), lambda i: (i, 0)),
            pl.BlockSpec((TILE_E, D), lambda i: (i, 0)),
            _full_spec((D, H * HID)),
            _full_spec((D, H * HID)),
            _full_spec((D, H * HID)),
            _full_spec((1, H * HID)),
            _full_spec((1, H * HID)),
            _full_spec((H * HID, 8)),
            _full_spec((D, D)),
            _full_spec((D, D)),
            _full_spec((1, D)),
            _full_spec((H, D)),
            _full_spec((8, D)),
        ],
        out_specs=[
            pl.BlockSpec((TILE_E, D), lambda i: (i, 0)),
            pl.BlockSpec((TILE_E, D), lambda i: (i, 0)),
        ],
        out_shape=[
            jax.ShapeDtypeStruct((E2, D), f32),
            jax.ShapeDtypeStruct((E2, D), f32),
        ],
    )(ps, pd, bridge_features, bf(p['a_Wsrc']), bf(p['a_Wtgt']), bf(p['a_Wbrg']),
      bsum_a, attn_a, blk, bf(p['a_Wmsg'][:D]), bf(p['a_Wmsg'][D:]),
      p['a_bmsg'].reshape(1, D), kron, tile16)

    zu = jnp.zeros((NPAD // SC_NS, D), f32)
    idx2d_1 = node_idx.reshape(SC_NW, -1, 80)
    idx2d_2 = pair_dst.reshape(SC_NW, -1, 80)
    u1, su1 = _scatter2_rows_sc(msgw1, exrow1, idx2d_1, zu)
    u2, su2 = _scatter2_rows_sc(msgw2, exrow2, idx2d_2, zu)

    grid_n = NPAD // TILE_N
    out = pl.pallas_call(
        _final_kernel,
        grid=(grid_n,),
        in_specs=[
            pl.BlockSpec((TILE_N, D), lambda i: (i, 0)),
            pl.BlockSpec((SC_NC, TILE_N, D), lambda i: (0, i, 0)),
            pl.BlockSpec((SC_NC, TILE_N, D), lambda i: (0, i, 0)),
            pl.BlockSpec((SC_NC, TILE_N, D), lambda i: (0, i, 0)),
            pl.BlockSpec((SC_NC, TILE_N, D), lambda i: (0, i, 0)),
            _full_spec((D, D)),
            _full_spec((1, D)),
            _full_spec((D, D)),
            _full_spec((1, D)),
            _full_spec((D, D)),
            _full_spec((1, D)),
            _full_spec((H, D)),
        ],
        out_specs=pl.BlockSpec((TILE_N, D), lambda i: (i, 0)),
        out_shape=jax.ShapeDtypeStruct((NPAD, D), f32),
    )(nh_pad, u1, su1, u2, su2, p['Wnp'], p['bnp'].reshape(1, D),
      p['i_Wout'], p['i_bout'].reshape(1, D),
      p['a_Wout'], p['a_bout'].reshape(1, D), kron)

    return out[:N]


# speedup vs baseline: 1.2237x; 1.0106x over previous
import functools
import jax
import jax.numpy as jnp
from jax import lax
from jax.experimental import pallas as pl
from jax.experimental.pallas import tpu as pltpu
from jax.experimental.pallas import tpu_sc as plsc

N = 10000
M = 2000
E = 320000
E2 = 320000
D = 128
H = 4
HID = 128
HD = D // H
NPAD = 10240
TILE_E = 512
TILE_N = 512


def _gelu(x):
    return 0.5 * x * (1.0 + lax.erf(x * 0.7071067811865476))


def _ln_kernel(x_ref, s_ref, b_ref, o_ref):
    x = x_ref[...]
    mu = jnp.mean(x, axis=-1, keepdims=True)
    var = jnp.mean((x - mu) ** 2, axis=-1, keepdims=True)
    o_ref[...] = (x - mu) / jnp.sqrt(var + 1e-5) * s_ref[...] + b_ref[...]


def _layer_norm_tc(x, scale, bias, tile):
    rows = x.shape[0]
    grid = rows // tile
    return pl.pallas_call(
        _ln_kernel,
        grid=(grid,),
        in_specs=[
            pl.BlockSpec((tile, D), lambda i: (i, 0)),
            pl.BlockSpec((1, D), lambda i: (0, 0)),
            pl.BlockSpec((1, D), lambda i: (0, 0)),
        ],
        out_specs=pl.BlockSpec((tile, D), lambda i: (i, 0)),
        out_shape=jax.ShapeDtypeStruct((rows, D), jnp.float32),
    )(x, scale.reshape(1, D), bias.reshape(1, D))


def _inter_edge_kernel(sc_ref, tc_ref, wsrc_ref, wtgt_ref, bsum_ref,
                       attn_ref, blk_ref, wmsg_ref, bmsg_ref, kron_ref,
                       tile16_ref, msgw_ref, exrow_ref):
    scv = sc_ref[...].astype(jnp.bfloat16)
    tcv = tc_ref[...].astype(jnp.bfloat16)
    h = (jnp.dot(scv, wsrc_ref[...], preferred_element_type=jnp.float32)
         + jnp.dot(tcv, wtgt_ref[...], preferred_element_type=jnp.float32)
         + bsum_ref[...])
    h = _gelu(h)
    logits = jnp.dot(h * attn_ref[...], blk_ref[...],
                     preferred_element_type=jnp.float32)
    ex = jnp.exp(logits)
    msg = jnp.dot(scv, wmsg_ref[...], preferred_element_type=jnp.float32) + bmsg_ref[...]
    scale = jnp.dot(ex[:, :H], kron_ref[...], preferred_element_type=jnp.float32)
    msgw_ref[...] = msg * scale
    exrow_ref[...] = jnp.dot(ex, tile16_ref[...], preferred_element_type=jnp.float32)


def _intra_edge_kernel(ps_ref, pd_ref, br_ref, wsrc_ref, wtgt_ref, wbrg_ref,
                       bsum_ref, attn_ref, blk_ref, wm1_ref, wm2_ref,
                       bmsg_ref, kron_ref, tile16_ref, msgw_ref, exrow_ref):
    psv = ps_ref[...].astype(jnp.bfloat16)
    brv = br_ref[...].astype(jnp.bfloat16)
    h = (jnp.dot(psv, wsrc_ref[...], preferred_element_type=jnp.float32)
         + jnp.dot(pd_ref[...].astype(jnp.bfloat16), wtgt_ref[...],
                   preferred_element_type=jnp.float32)
         + jnp.dot(brv, wbrg_ref[...], preferred_element_type=jnp.float32)
         + bsum_ref[...])
    h = _gelu(h)
    logits = jnp.dot(h * attn_ref[...], blk_ref[...],
                     preferred_element_type=jnp.float32)
    ex = jnp.exp(logits)
    msg = (jnp.dot(psv, wm1_ref[...], preferred_element_type=jnp.float32)
           + jnp.dot(brv, wm2_ref[...], preferred_element_type=jnp.float32)
           + bmsg_ref[...])
    scale = jnp.dot(ex[:, :H], kron_ref[...], preferred_element_type=jnp.float32)
    msgw_ref[...] = msg * scale
    exrow_ref[...] = jnp.dot(ex, tile16_ref[...], preferred_element_type=jnp.float32)


def _final_kernel(nh_ref, u1_ref, su1_ref, u2_ref, su2_ref, wnp_ref, bnp_ref,
                  wo1_ref, bo1_ref, wo2_ref, bo2_ref, kron_ref, o_ref):
    kron = kron_ref[...]
    s1 = (su1_ref[0] + su1_ref[1])[:, :H]
    s2 = (su2_ref[0] + su2_ref[1])[:, :H]
    d1 = jnp.dot(s1, kron, preferred_element_type=jnp.float32) + 1e-16
    d2 = jnp.dot(s2, kron, preferred_element_type=jnp.float32) + 1e-16
    g1 = (u1_ref[0] + u1_ref[1]) / d1
    g2 = (u2_ref[0] + u2_ref[1]) / d2
    out = (jnp.dot(nh_ref[...], wnp_ref[...], preferred_element_type=jnp.float32)
           + bnp_ref[...]
           + jnp.dot(g1, wo1_ref[...], preferred_element_type=jnp.float32)
           + bo1_ref[...]
           + jnp.dot(g2, wo2_ref[...], preferred_element_type=jnp.float32)
           + bo2_ref[...])
    o_ref[...] = out


SC_NC = 2
SC_NS = 16
SC_NW = SC_NC * SC_NS


def _gather_rows_sc(table, idx):
    B = idx.shape[0]
    b_per_w = B // SC_NW
    GB = 400
    nblk = b_per_w // GB
    mesh = plsc.VectorSubcoreMesh(core_axis_name="c", subcore_axis_name="s",
                                  num_cores=SC_NC, num_subcores=SC_NS)

    @functools.partial(
        pl.kernel, mesh=mesh,
        out_type=jax.ShapeDtypeStruct((B, D), jnp.float32),
        scratch_types=[
            pltpu.VMEM((b_per_w,), jnp.int32),
            pltpu.VMEM((GB, D), jnp.float32),
            pltpu.VMEM((GB, D), jnp.float32),
            pltpu.SemaphoreType.DMA,
            pltpu.SemaphoreType.DMA,
        ],
    )
    def k(table_hbm, idx_hbm, out_hbm, idx_v, rows_a, rows_b, sem_a, sem_b):
        wid = lax.axis_index("s") * SC_NC + lax.axis_index("c")
        base = wid * b_per_w
        pltpu.sync_copy(idx_hbm.at[pl.ds(base, b_per_w)], idx_v)
        pltpu.async_copy(table_hbm.at[idx_v.at[pl.ds(0, GB)]],
                         rows_a, sem_a).wait()

        def pair_body(i, carry):
            j0 = 2 * i
            cb = pltpu.async_copy(
                table_hbm.at[idx_v.at[pl.ds((j0 + 1) * GB, GB)]], rows_b, sem_b)
            pltpu.sync_copy(rows_a, out_hbm.at[pl.ds(base + j0 * GB, GB)])
            cb.wait()
            ca = pltpu.async_copy(
                table_hbm.at[idx_v.at[pl.ds((j0 + 2) * GB, GB)]], rows_a, sem_a)
            pltpu.sync_copy(rows_b, out_hbm.at[pl.ds(base + (j0 + 1) * GB, GB)])
            ca.wait()
            return carry

        lax.fori_loop(0, (nblk - 1) // 2, pair_body, 0)
        pltpu.sync_copy(rows_a, out_hbm.at[pl.ds(base + (nblk - 1) * GB, GB)])

    return k(table, idx)


def _scatter2_rows_sc(msgw, exrow, idx2d, zeros_u):
    B = msgw.shape[0]
    b_per_w = B // SC_NW
    RB = 80
    nblk = b_per_w // RB
    npair = (nblk - 1) // 2
    rows_per_s = NPAD // SC_NS
    mesh = plsc.VectorSubcoreMesh(core_axis_name="c", subcore_axis_name="s",
                                  num_cores=SC_NC, num_subcores=SC_NS)
    part_t = jax.ShapeDtypeStruct((SC_NC, NPAD, D), jnp.float32)

    @functools.partial(
        pl.kernel, mesh=mesh,
        out_type=[part_t, part_t],
        scratch_types=[
            pltpu.VMEM((nblk, RB), jnp.int32),
            pltpu.VMEM((RB, D), jnp.float32),
            pltpu.VMEM((RB, D), jnp.float32),
            pltpu.VMEM_SHARED((NPAD, D), jnp.float32),
            pltpu.SemaphoreType.DMA,
            pltpu.SemaphoreType.DMA,
        ],
    )
    def k(msgw_hbm, exrow_hbm, idx2d_hbm, zu_hbm, u_out, s_out,
          idx2_v, m_a, m_b, uacc, sem_a, sem_b):
        cid = lax.axis_index("c")
        sid = lax.axis_index("s")
        wid = sid * SC_NC + cid
        base = wid * b_per_w
        my_rows = pl.ds(sid * rows_per_s, rows_per_s)

        pltpu.sync_copy(idx2d_hbm.at[wid], idx2_v)
        pltpu.sync_copy(zu_hbm, uacc.at[my_rows])
        plsc.subcore_barrier()

        def scatter_stream(rows_hbm):
            pltpu.async_copy(rows_hbm.at[pl.ds(base, RB)], m_a, sem_a).wait()

            def pair_body(i, carry):
                j0 = 2 * i
                cb = pltpu.async_copy(
                    rows_hbm.at[pl.ds(base + (j0 + 1) * RB, RB)], m_b, sem_b)
                pltpu.sync_copy(m_a, uacc.at[idx2_v.at[j0]], add=True)
                cb.wait()
                ca = pltpu.async_copy(
                    rows_hbm.at[pl.ds(base + (j0 + 2) * RB, RB)], m_a, sem_a)
                pltpu.sync_copy(m_b, uacc.at[idx2_v.at[j0 + 1]], add=True)
                ca.wait()
                return carry

            lax.fori_loop(0, npair, pair_body, 0)
            pltpu.sync_copy(m_a, uacc.at[idx2_v.at[nblk - 1]], add=True)

        scatter_stream(msgw_hbm)
        plsc.subcore_barrier()
        pltpu.sync_copy(uacc.at[my_rows], u_out.at[cid, my_rows])
        pltpu.sync_copy(zu_hbm, uacc.at[my_rows])
        plsc.subcore_barrier()

        scatter_stream(exrow_hbm)
        plsc.subcore_barrier()
        pltpu.sync_copy(uacc.at[my_rows], s_out.at[cid, my_rows])

    return k(msgw, exrow, idx2d, zeros_u)


def _full_spec(shape):
    nd = len(shape)
    return pl.BlockSpec(shape, lambda i: (0,) * nd)


def kernel(node_features, edge_features, bridge_features, params, node_idx, edge_idx, pair_src, pair_dst):
    p = params
    f32 = jnp.float32
    bf = lambda a: a.astype(jnp.bfloat16)

    hid_ids = jnp.arange(H * HID, dtype=jnp.int32) // HID
    blk = (hid_ids[:, None] == jnp.arange(8, dtype=jnp.int32)[None, :]).astype(f32)
    hd_ids = jnp.arange(D, dtype=jnp.int32) // HD
    kron = (jnp.arange(H, dtype=jnp.int32)[:, None] == hd_ids[None, :]).astype(f32)
    col_mod8 = jnp.arange(D, dtype=jnp.int32) % 8
    tile16 = (jnp.arange(8, dtype=jnp.int32)[:, None] == col_mod8[None, :]).astype(f32)

    nf_pad = jnp.pad(node_features, ((0, NPAD - N), (0, 0)))
    nh_pad = _layer_norm_tc(nf_pad, p['ln_n_s'], p['ln_n_b'], 512)
    nh = nh_pad[:N]
    eh = _layer_norm_tc(edge_features, p['ln_e_s'], p['ln_e_b'], 400)

    sc_i = _gather_rows_sc(eh, edge_idx)
    tc_i = _gather_rows_sc(nh, node_idx)
    ps = _gather_rows_sc(nh, pair_src)
    pd = _gather_rows_sc(nh, pair_dst)

    bsum_i = (p['i_bsrc'] + p['i_btgt']).reshape(1, H * HID)
    attn_i = p['i_attn'].reshape(1, H * HID)
    grid_e = E // TILE_E
    msgw1, exrow1 = pl.pallas_call(
        _inter_edge_kernel,
        grid=(grid_e,),
        in_specs=[
            pl.BlockSpec((TILE_E, D), lambda i: (i, 0)),
            pl.BlockSpec((TILE_E, D), lambda i: (i, 0)),
            _full_spec((D, H * HID)),
            _full_spec((D, H * HID)),
            _full_spec((1, H * HID)),
            _full_spec((1, H * HID)),
            _full_spec((H * HID, 8)),
            _full_spec((D, D)),
            _full_spec((1, D)),
            _full_spec((H, D)),
            _full_spec((8, D)),
        ],
        out_specs=[
            pl.BlockSpec((TILE_E, D), lambda i: (i, 0)),
            pl.BlockSpec((TILE_E, D), lambda i: (i, 0)),
        ],
        out_shape=[
            jax.ShapeDtypeStruct((E, D), f32),
            jax.ShapeDtypeStruct((E, D), f32),
        ],
    )(sc_i, tc_i, bf(p['i_Wsrc']), bf(p['i_Wtgt']), bsum_i, attn_i, blk,
      bf(p['i_Wmsg']), p['i_bmsg'].reshape(1, D), kron, tile16)

    bsum_a = (p['a_bsrc'] + p['a_btgt'] + p['a_bbrg']).reshape(1, H * HID)
    attn_a = p['a_attn'].reshape(1, H * HID)
    grid_e2 = E2 // TILE_E
    msgw2, exrow2 = pl.pallas_call(
        _intra_edge_kernel,
        grid=(grid_e2,),
        in_specs=[
            pl.BlockSpec((TILE_E, D), lambda i: (i, 0)),
            pl.BlockSpec((TILE_E, D), lambda i: (i, 0)),
            pl.BlockSpec((TILE_E, D), lambda i: (i, 0)),
            _full_spec((D, H * HID)),
            _full_spec((D, H * HID)),
            _full_spec((D, H * HID)),
            _full_spec((1, H * HID)),
            _full_spec((1, H * HID)),
            _full_spec((H * HID, 8)),
            _full_spec((D, D)),
            _full_spec((D, D)),
            _full_spec((1, D)),
            _full_spec((H, D)),
            _full_spec((8, D)),
        ],
        out_specs=[
            pl.BlockSpec((TILE_E, D), lambda i: (i, 0)),
            pl.BlockSpec((TILE_E, D), lambda i: (i, 0)),
        ],
        out_shape=[
            jax.ShapeDtypeStruct((E2, D), f32),
            jax.ShapeDtypeStruct((E2, D), f32),
        ],
    )(ps, pd, bridge_features, bf(p['a_Wsrc']), bf(p['a_Wtgt']), bf(p['a_Wbrg']),
      bsum_a, attn_a, blk, bf(p['a_Wmsg'][:D]), bf(p['a_Wmsg'][D:]),
      p['a_bmsg'].reshape(1, D), kron, tile16)

    zu = jnp.zeros((NPAD // SC_NS, D), f32)
    idx2d_1 = node_idx.reshape(SC_NW, -1, 80)
    idx2d_2 = pair_dst.reshape(SC_NW, -1, 80)
    u1, su1 = _scatter2_rows_sc(msgw1, exrow1, idx2d_1, zu)
    u2, su2 = _scatter2_rows_sc(msgw2, exrow2, idx2d_2, zu)

    grid_n = NPAD // TILE_N
    out = pl.pallas_call(
        _final_kernel,
        grid=(grid_n,),
        in_specs=[
            pl.BlockSpec((TILE_N, D), lambda i: (i, 0)),
            pl.BlockSpec((SC_NC, TILE_N, D), lambda i: (0, i, 0)),
            pl.BlockSpec((SC_NC, TILE_N, D), lambda i: (0, i, 0)),
            pl.BlockSpec((SC_NC, TILE_N, D), lambda i: (0, i, 0)),
            pl.BlockSpec((SC_NC, TILE_N, D), lambda i: (0, i, 0)),
            _full_spec((D, D)),
            _full_spec((1, D)),
            _full_spec((D, D)),
            _full_spec((1, D)),
            _full_spec((D, D)),
            _full_spec((1, D)),
            _full_spec((H, D)),
        ],
        out_specs=pl.BlockSpec((TILE_N, D), lambda i: (i, 0)),
        out_shape=jax.ShapeDtypeStruct((NPAD, D), f32),
    )(nh_pad, u1, su1, u2, su2, p['Wnp'], p['bnp'].reshape(1, D),
      p['i_Wout'], p['i_bout'].reshape(1, D),
      p['a_Wout'], p['a_bout'].reshape(1, D), kron)

    return out[:N]
